# baseline (speedup 1.0000x reference)
import functools

import jax
import jax.numpy as jnp
from jax import lax
from jax.experimental import pallas as pl
from jax.experimental.pallas import tpu as pltpu
from jax.experimental.pallas import tpu_sc as plsc

N_NODES = 10000
N_EDGES = 320000
N_GRAPHS = 16
HID = 64
BN = 1000
BE = 2000
NW = 32
EPW = N_EDGES // NW
CB = 80
NCH = EPW // CB
DSTR = 1000
NDW = N_NODES // DSTR

_f32 = jnp.float32


def _sel_rows(b, tab, n_cols):
    acc = jnp.zeros((BN, n_cols), _f32)
    for g in range(N_GRAPHS):
        sel = (b == g).astype(_f32)
        acc += sel * tab[g:g + 1, :]
    return acc


def _tables_body(x_ref, b_ref, ss_ref, wa_ref, wb_ref, wua_ref, wub_ref,
                 ca_ref, u_ref, ta_ref, tb_ref, xb_ref):
    x = x_ref[...] * ss_ref[0:1, :] + ss_ref[1:2, :]
    b = b_ref[...]
    uA = jnp.dot(u_ref[...], wua_ref[...], preferred_element_type=_f32)
    uB = jnp.dot(u_ref[...], wub_ref[...], preferred_element_type=_f32)
    z16 = jnp.zeros((N_GRAPHS, HID), _f32)
    UA = jnp.concatenate([z16, uA], axis=1) + ca_ref[...]
    UB = jnp.concatenate([uB, z16], axis=1)
    ta_ref[...] = (jnp.dot(x, wa_ref[...], preferred_element_type=_f32)
                   + _sel_rows(b, UA, 128))
    tb_ref[...] = (jnp.dot(x, wb_ref[...], preferred_element_type=_f32)
                   + _sel_rows(b, UB, 128))
    xb_ref[...] = x


def _tables_tc(x_raw, batch2d, ss, WA_T, WB_T, WuA_T, WuB_T, cA, u):
    Fx = x_raw.shape[1]
    Fu = u.shape[1]
    full = lambda s: pl.BlockSpec(s, lambda i: (0, 0))
    return pl.pallas_call(
        _tables_body,
        grid=(N_NODES // BN,),
        in_specs=[
            pl.BlockSpec((BN, Fx), lambda i: (i, 0)),
            pl.BlockSpec((BN, 1), lambda i: (i, 0)),
            full((2, Fx)), full((Fx, 128)), full((Fx, 128)),
            full((Fu, HID)), full((Fu, HID)), full((1, 128)),
            full((N_GRAPHS, Fu)),
        ],
        out_specs=[
            pl.BlockSpec((BN, 128), lambda i: (i, 0)),
            pl.BlockSpec((BN, 128), lambda i: (i, 0)),
            pl.BlockSpec((BN, Fx), lambda i: (i, 0)),
        ],
        out_shape=[
            jax.ShapeDtypeStruct((N_NODES, 128), _f32),
            jax.ShapeDtypeStruct((N_NODES, 128), _f32),
            jax.ShapeDtypeStruct((N_NODES, Fx), _f32),
        ],
    )(x_raw, batch2d, ss, WA_T, WB_T, WuA_T, WuB_T, cA, u)


def _edge_body(act, g_ref, e_ref, bc_ref, es_ref, we_ref, wn_ref, mm_ref,
               me_ref, st_ref, ea_ref):
    g = g_ref[...]
    e = e_ref[...] * es_ref[0:1, :] + es_ref[1:2, :]
    ep = g[:, :HID] + jnp.dot(e, we_ref[...], preferred_element_type=_f32)
    en = jnp.maximum(ep, 0.0) if act else ep
    m = g[:, HID:] + jnp.dot(en, wn_ref[...], preferred_element_type=_f32)
    if act:
        m = jnp.maximum(m, 0.0)
    mm_ref[...] = m
    me_ref[...] = en
    bc = bc_ref[...]

    @pl.when(pl.program_id(0) == 0)
    def _():
        st_ref[...] = jnp.zeros_like(st_ref)
        ea_ref[...] = jnp.zeros_like(ea_ref)
    rows = [jnp.sum((bc == g_).astype(_f32) * en, axis=0)[None, :]
            for g_ in range(N_GRAPHS)]
    ea_ref[...] += jnp.concatenate(rows, axis=0)
    if act:
        st_ref[...] += jnp.concatenate([jnp.sum(en, 0)[None, :],
                                        jnp.sum(en * en, 0)[None, :]], axis=0)


def _edge_tc(G, Eprev, batchcol2d, ss_e, We_T, Wn_T, act):
    Fe = We_T.shape[0]
    full = lambda s: pl.BlockSpec(s, lambda i: (0, 0))
    return pl.pallas_call(
        functools.partial(_edge_body, act),
        grid=(N_EDGES // BE,),
        in_specs=[
            pl.BlockSpec((BE, 128), lambda i: (i, 0)),
            pl.BlockSpec((BE, Fe), lambda i: (i, 0)),
            pl.BlockSpec((BE, 1), lambda i: (i, 0)),
            full((2, Fe)), full((Fe, HID)), full((HID, HID)),
        ],
        out_specs=[pl.BlockSpec((BE, HID), lambda i: (i, 0)),
                   pl.BlockSpec((BE, HID), lambda i: (i, 0)),
                   pl.BlockSpec((2, HID), lambda i: (0, 0)),
                   pl.BlockSpec((N_GRAPHS, HID), lambda i: (0, 0))],
        out_shape=[jax.ShapeDtypeStruct((N_EDGES, HID), _f32),
                   jax.ShapeDtypeStruct((N_EDGES, HID), _f32),
                   jax.ShapeDtypeStruct((2, HID), _f32),
                   jax.ShapeDtypeStruct((N_GRAPHS, HID), _f32)],
    )(G, Eprev, batchcol2d, ss_e, We_T, Wn_T)


def _node_body(act, xb_ref, cm_ref, b_ref,
               w2x_ref, w2a_ref, w2u_ref, u_ref, bn2_ref, xr_ref, na_ref,
               st_ref):
    agg = cm_ref[...]
    b = b_ref[...]
    uW = jnp.dot(u_ref[...], w2u_ref[...], preferred_element_type=_f32)
    xp = (jnp.dot(xb_ref[...], w2x_ref[...], preferred_element_type=_f32)
          + jnp.dot(agg, w2a_ref[...], preferred_element_type=_f32)
          + _sel_rows(b, uW, HID) + bn2_ref[...])
    xn = jnp.maximum(xp, 0.0) if act else xp
    xr_ref[...] = xn

    def seg16(v):
        rows = [jnp.sum((b == g).astype(_f32) * v, axis=0)[None, :]
                for g in range(N_GRAPHS)]
        return jnp.concatenate(rows, axis=0)

    @pl.when(pl.program_id(0) == 0)
    def _():
        na_ref[...] = jnp.zeros_like(na_ref)
        st_ref[...] = jnp.zeros_like(st_ref)
    na_ref[...] += seg16(xn)
    if act:
        st_ref[...] += jnp.concatenate([jnp.sum(xn, 0)[None, :],
                                        jnp.sum(xn * xn, 0)[None, :]], axis=0)


def _node_tc(xb, cm, batch2d, W2x_T, W2a_T, W2u_T, u, bn2, act):
    Fx = xb.shape[1]
    Fu = u.shape[1]
    full = lambda s: pl.BlockSpec(s, lambda i: (0, 0))
    return pl.pallas_call(
        functools.partial(_node_body, act),
        grid=(N_NODES // BN,),
        in_specs=[
            pl.BlockSpec((BN, Fx), lambda i: (i, 0)),
            pl.BlockSpec((BN, HID), lambda i: (i, 0)),
            pl.BlockSpec((BN, 1), lambda i: (i, 0)),
            full((Fx, HID)), full((HID, HID)), full((Fu, HID)),
            full((N_GRAPHS, Fu)), full((1, HID)),
        ],
        out_specs=[
            pl.BlockSpec((BN, HID), lambda i: (i, 0)),
            pl.BlockSpec((N_GRAPHS, HID), lambda i: (0, 0)),
            pl.BlockSpec((2, HID), lambda i: (0, 0)),
        ],
        out_shape=[
            jax.ShapeDtypeStruct((N_NODES, HID), _f32),
            jax.ShapeDtypeStruct((N_GRAPHS, HID), _f32),
            jax.ShapeDtypeStruct((2, HID), _f32),
        ],
    )(xb, cm, batch2d, W2x_T, W2a_T, W2u_T, u, bn2)


def _global_body(act, na_ref, ea_ref, u_ref, wg_ref, bg_ref, gam_ref,
                 bet_ref, un_ref):
    gin = jnp.concatenate([na_ref[...], ea_ref[...], u_ref[...]], axis=1)
    up = jnp.dot(gin, wg_ref[...], preferred_element_type=_f32) + bg_ref[...]
    if act:
        up = jnp.maximum(up, 0.0)
        m = jnp.mean(up, axis=0, keepdims=True)
        d = up - m
        v = jnp.mean(d * d, axis=0, keepdims=True)
        up = d * lax.rsqrt(v + 1e-5) * gam_ref[...] + bet_ref[...]
    un_ref[...] = up


def _global_tc(nagg, eagg, u, Wg_T, bg, gam, bet, act):
    Fu = u.shape[1]
    gdim = Wg_T.shape[0]
    full = lambda s: pl.BlockSpec(s, lambda i: (0, 0))
    return pl.pallas_call(
        functools.partial(_global_body, act),
        grid=(1,),
        in_specs=[full((N_GRAPHS, HID)), full((N_GRAPHS, HID)),
                  full((N_GRAPHS, Fu)), full((gdim, HID)), full((1, HID)),
                  full((1, HID)), full((1, HID))],
        out_specs=full((N_GRAPHS, HID)),
        out_shape=jax.ShapeDtypeStruct((N_GRAPHS, HID), _f32),
    )(nagg, eagg, u, Wg_T, bg, gam, bet)


def _sc_gather_body(ta_hbm, tb_hbm, col_hbm, row_hbm, g_hbm, ic, ir, ba, bb,
                    sa, sb):
    wid = lax.axis_index("s") * 2 + lax.axis_index("c")
    base = wid * EPW
    pltpu.sync_copy(col_hbm.at[wid], ic)
    pltpu.sync_copy(row_hbm.at[wid], ir)

    def chunk(j, carry):
        cpa = pltpu.async_copy(ta_hbm.at[ic.at[j]], ba, sa)
        cpb = pltpu.async_copy(tb_hbm.at[ir.at[j]], bb, sb)
        cpa.wait()
        cpb.wait()

        def addrow(r, c2):
            for k in range(8):
                sl = pl.ds(k * 16, 16)
                ba[r, sl] = ba[r, sl] + bb[r, sl]
            return c2
        lax.fori_loop(0, CB, addrow, 0, unroll=4)
        pltpu.sync_copy(ba, g_hbm.at[pl.ds(base + j * CB, CB)])
        return carry
    lax.fori_loop(0, NCH, chunk, 0)


def _sc_scatter_body(mm_hbm, me_hbm, col_hbm, seq_hbm, zr_hbm, cp_hbm,
                     ic80, iz80, mbm, mbe, sem, cshm, cshe):
    cc = lax.axis_index("c")
    ss = lax.axis_index("s")
    wid = ss * 2 + cc
    base = wid * EPW
    nzc = N_NODES // CB

    pltpu.sync_copy(zr_hbm, mbm)

    def zchunk(k, c2):
        c = jnp.minimum(ss * 8 + k, nzc - 1)
        pltpu.sync_copy(seq_hbm.at[c], iz80)
        pltpu.sync_copy(mbm, cshm.at[iz80])
        pltpu.sync_copy(mbm, cshe.at[iz80])
        return c2
    lax.fori_loop(0, 8, zchunk, 0)
    plsc.subcore_barrier()

    def chunk(j, carry):
        pltpu.sync_copy(col_hbm.at[wid, j], ic80)
        pltpu.sync_copy(mm_hbm.at[pl.ds(base + j * CB, CB)], mbm)
        pltpu.sync_copy(me_hbm.at[pl.ds(base + j * CB, CB)], mbe)
        pltpu.sync_copy(mbm, cshm.at[ic80], add=True)
        pltpu.sync_copy(mbe, cshe.at[ic80], add=True)
        return carry
    lax.fori_loop(0, NCH, chunk, 0)
    plsc.subcore_barrier()

    def dchunk(k, c2):
        c = jnp.minimum(ss * 8 + k, nzc - 1)
        sl = pl.ds(c * CB, CB)
        pltpu.sync_copy(seq_hbm.at[c], iz80)
        pltpu.async_copy(cshm.at[iz80], mbm, sem).wait()
        pltpu.sync_copy(mbm, cp_hbm.at[cc, 0, sl])
        pltpu.async_copy(cshe.at[iz80], mbe, sem).wait()
        pltpu.sync_copy(mbe, cp_hbm.at[cc, 1, sl])
        return c2
    lax.fori_loop(0, 8, dchunk, 0)


@functools.cache
def _sc_kernels():
    mesh = plsc.VectorSubcoreMesh(core_axis_name="c", subcore_axis_name="s")
    gather = functools.partial(
        pl.kernel, mesh=mesh,
        out_type=jax.ShapeDtypeStruct((N_EDGES, 128), _f32),
        scratch_types=[
            pltpu.VMEM((NCH, CB), jnp.int32),
            pltpu.VMEM((NCH, CB), jnp.int32),
            pltpu.VMEM((CB, 128), _f32),
            pltpu.VMEM((CB, 128), _f32),
            pltpu.SemaphoreType.DMA,
            pltpu.SemaphoreType.DMA,
        ])(_sc_gather_body)
    scatter = functools.partial(
        pl.kernel, mesh=mesh,
        out_type=jax.ShapeDtypeStruct((2, 2, N_NODES, HID), _f32),
        scratch_types=[
            pltpu.VMEM((CB,), jnp.int32),
            pltpu.VMEM((CB,), jnp.int32),
            pltpu.VMEM((CB, HID), _f32),
            pltpu.VMEM((CB, HID), _f32),
            pltpu.SemaphoreType.DMA,
            pltpu.VMEM_SHARED((N_NODES, HID), _f32),
            pltpu.VMEM_SHARED((N_NODES, HID), _f32),
        ])(_sc_scatter_body)
    return gather, scatter


def _gather_add(TA, TB, col3, row3):
    return _sc_kernels()[0](TA, TB, col3, row3)


def _scatter_m(Mm, col3):
    return jax.ops.segment_sum(Mm, col3.reshape(-1), num_segments=N_NODES)


def kernel(x, edge_index, edge_attr, u, batch, params):
    row3 = edge_index[0].astype(jnp.int32).reshape(NW, NCH, CB)
    col3 = edge_index[1].astype(jnp.int32).reshape(NW, NCH, CB)
    batch2d = batch.astype(jnp.int32).reshape(N_NODES, 1)
    batchcol2d = batch.astype(jnp.int32)[edge_index[1]].reshape(N_EDGES, 1)

    e_scale = None
    e_shift = None
    ss_x = jnp.concatenate([jnp.ones((1, x.shape[1]), _f32),
                            jnp.zeros((1, x.shape[1]), _f32)], axis=0)
    x_raw = x
    u_cur = u
    Eprev = edge_attr
    L = len(params)
    x_out = e_out = None

    for i, p in enumerate(params):
        act = i != L - 1
        Fx = x_raw.shape[1]
        Fe = Eprev.shape[1] if i == 0 else HID
        We, be = p['We'], p['be']
        Wn1, bn1 = p['Wn1'], p['bn1']
        Wn2, bn2 = p['Wn2'], p['bn2']
        Wg, bg = p['Wg'], p['bg']
        We_d, We_s = We[:, :Fx], We[:, Fx:2 * Fx]
        We_e, We_u = We[:, 2 * Fx:2 * Fx + Fe], We[:, 2 * Fx + Fe:]
        Wn1_d, Wn1_s = Wn1[:, :Fx], Wn1[:, Fx:2 * Fx]
        Wn1_e, Wn1_u = Wn1[:, 2 * Fx:2 * Fx + HID], Wn1[:, 2 * Fx + HID:]
        if e_scale is None:
            ss_e = jnp.concatenate([jnp.ones((1, Fe), _f32),
                                    jnp.zeros((1, Fe), _f32)], axis=0)
        else:
            ss_e = jnp.concatenate([e_scale[None, :], e_shift[None, :]],
                                   axis=0)
        WA_T = jnp.concatenate([We_d, Wn1_d], axis=0).T
        WB_T = jnp.concatenate([We_s, Wn1_s], axis=0).T
        cA = jnp.concatenate([be, bn1])[None, :]

        TA, TB, xb = _tables_tc(x_raw, batch2d, ss_x, WA_T, WB_T,
                                Wn1_u.T, We_u.T, cA, u_cur)
        G = _gather_add(TA, TB, col3, row3)
        Mm, Me, estats, eagg = _edge_tc(G, Eprev, batchcol2d, ss_e,
                                        We_e.T, Wn1_e.T, act)
        Cm = _scatter_m(Mm, col3)
        x_raw, nagg, xstats = _node_tc(
            xb, Cm, batch2d,
            Wn2[:, :Fx].T, Wn2[:, Fx:Fx + HID].T, Wn2[:, Fx + HID:].T,
            u_cur, bn2[None, :], act)
        u_cur = _global_tc(nagg, eagg, u_cur, Wg.T, bg[None, :],
                           (p['gam_g'] if act else bg)[None, :],
                           (p['bet_g'] if act else bg)[None, :], act)
        if act:
            mx = xstats[0] / N_NODES
            vx = xstats[1] / N_NODES - mx * mx
            sx = p['gam_n'] / jnp.sqrt(vx + 1e-5)
            ss_x = jnp.concatenate([sx[None, :], (p['bet_n'] - mx * sx)[None, :]],
                                   axis=0)
            me = estats[0] / N_EDGES
            ve = estats[1] / N_EDGES - me * me
            e_scale = p['gam_e'] / jnp.sqrt(ve + 1e-5)
            e_shift = p['bet_e'] - me * e_scale
            Eprev = Me
        else:
            x_out = x_raw
            e_out = Me

    return (x_out, e_out, u_cur)


# speedup vs baseline: 1.1732x; 1.1732x over previous
import functools

import jax
import jax.numpy as jnp
from jax import lax
from jax.experimental import pallas as pl
from jax.experimental.pallas import tpu as pltpu
from jax.experimental.pallas import tpu_sc as plsc

N_NODES = 10000
N_EDGES = 320000
N_GRAPHS = 16
HID = 64
BN = 1000
BE = 2000
NW = 32
EPW = N_EDGES // NW
CB = 80
NCH = EPW // CB
DSTR = 1000
NDW = N_NODES // DSTR

_f32 = jnp.float32


def _sel_rows(b, tab, n_cols):
    acc = jnp.zeros((BN, n_cols), _f32)
    for g in range(N_GRAPHS):
        sel = (b == g).astype(_f32)
        acc += sel * tab[g:g + 1, :]
    return acc


def _tables_body(x_ref, b_ref, ss_ref, wa_ref, wb_ref, wua_ref, wub_ref,
                 ca_ref, u_ref, ta_ref, tb_ref, xb_ref):
    x = x_ref[...] * ss_ref[0:1, :] + ss_ref[1:2, :]
    b = b_ref[...]
    uA = jnp.dot(u_ref[...], wua_ref[...], preferred_element_type=_f32)
    uB = jnp.dot(u_ref[...], wub_ref[...], preferred_element_type=_f32)
    z16 = jnp.zeros((N_GRAPHS, HID), _f32)
    UA = jnp.concatenate([z16, uA], axis=1) + ca_ref[...]
    UB = jnp.concatenate([uB, z16], axis=1)
    ta_ref[...] = (jnp.dot(x, wa_ref[...], preferred_element_type=_f32)
                   + _sel_rows(b, UA, 128))
    tb_ref[...] = (jnp.dot(x, wb_ref[...], preferred_element_type=_f32)
                   + _sel_rows(b, UB, 128))
    xb_ref[...] = x


def _tables_tc(x_raw, batch2d, ss, WA_T, WB_T, WuA_T, WuB_T, cA, u):
    Fx = x_raw.shape[1]
    Fu = u.shape[1]
    full = lambda s: pl.BlockSpec(s, lambda i: (0, 0))
    return pl.pallas_call(
        _tables_body,
        grid=(N_NODES // BN,),
        in_specs=[
            pl.BlockSpec((BN, Fx), lambda i: (i, 0)),
            pl.BlockSpec((BN, 1), lambda i: (i, 0)),
            full((2, Fx)), full((Fx, 128)), full((Fx, 128)),
            full((Fu, HID)), full((Fu, HID)), full((1, 128)),
            full((N_GRAPHS, Fu)),
        ],
        out_specs=[
            pl.BlockSpec((BN, 128), lambda i: (i, 0)),
            pl.BlockSpec((BN, 128), lambda i: (i, 0)),
            pl.BlockSpec((BN, Fx), lambda i: (i, 0)),
        ],
        out_shape=[
            jax.ShapeDtypeStruct((N_NODES, 128), _f32),
            jax.ShapeDtypeStruct((N_NODES, 128), _f32),
            jax.ShapeDtypeStruct((N_NODES, Fx), _f32),
        ],
    )(x_raw, batch2d, ss, WA_T, WB_T, WuA_T, WuB_T, cA, u)


def _edge_body(act, g_ref, e_ref, es_ref, we_ref, wn_ref, mm_ref, me_ref,
               st_ref):
    g = g_ref[...]
    e = e_ref[...] * es_ref[0:1, :] + es_ref[1:2, :]
    ep = g[:, :HID] + jnp.dot(e, we_ref[...], preferred_element_type=_f32)
    en = jnp.maximum(ep, 0.0) if act else ep
    m = g[:, HID:] + jnp.dot(en, wn_ref[...], preferred_element_type=_f32)
    if act:
        m = jnp.maximum(m, 0.0)
    mm_ref[...] = m
    me_ref[...] = en
    if act:
        @pl.when(pl.program_id(0) == 0)
        def _():
            st_ref[...] = jnp.zeros_like(st_ref)
        st_ref[...] += jnp.concatenate([jnp.sum(en, 0)[None, :],
                                        jnp.sum(en * en, 0)[None, :]], axis=0)


def _edge_tc(G, Eprev, ss_e, We_T, Wn_T, act):
    Fe = We_T.shape[0]
    full = lambda s: pl.BlockSpec(s, lambda i: (0, 0))
    return pl.pallas_call(
        functools.partial(_edge_body, act),
        grid=(N_EDGES // BE,),
        in_specs=[
            pl.BlockSpec((BE, 128), lambda i: (i, 0)),
            pl.BlockSpec((BE, Fe), lambda i: (i, 0)),
            full((2, Fe)), full((Fe, HID)), full((HID, HID)),
        ],
        out_specs=[pl.BlockSpec((BE, HID), lambda i: (i, 0)),
                   pl.BlockSpec((BE, HID), lambda i: (i, 0)),
                   pl.BlockSpec((2, HID), lambda i: (0, 0))],
        out_shape=[jax.ShapeDtypeStruct((N_EDGES, HID), _f32),
                   jax.ShapeDtypeStruct((N_EDGES, HID), _f32),
                   jax.ShapeDtypeStruct((2, HID), _f32)],
    )(G, Eprev, ss_e, We_T, Wn_T)


def _node_body(act, xb_ref, c0m_ref, c1m_ref, c0e_ref, c1e_ref, b_ref,
               w2x_ref, w2a_ref, w2u_ref, u_ref, bn2_ref, xr_ref, na_ref,
               ea_ref, st_ref):
    agg = c0m_ref[...] + c1m_ref[...]
    en = c0e_ref[...] + c1e_ref[...]
    b = b_ref[...]
    uW = jnp.dot(u_ref[...], w2u_ref[...], preferred_element_type=_f32)
    xp = (jnp.dot(xb_ref[...], w2x_ref[...], preferred_element_type=_f32)
          + jnp.dot(agg, w2a_ref[...], preferred_element_type=_f32)
          + _sel_rows(b, uW, HID) + bn2_ref[...])
    xn = jnp.maximum(xp, 0.0) if act else xp
    xr_ref[...] = xn

    def seg16(v):
        rows = [jnp.sum((b == g).astype(_f32) * v, axis=0)[None, :]
                for g in range(N_GRAPHS)]
        return jnp.concatenate(rows, axis=0)

    @pl.when(pl.program_id(0) == 0)
    def _():
        na_ref[...] = jnp.zeros_like(na_ref)
        ea_ref[...] = jnp.zeros_like(ea_ref)
        st_ref[...] = jnp.zeros_like(st_ref)
    na_ref[...] += seg16(xn)
    ea_ref[...] += seg16(en)
    if act:
        st_ref[...] += jnp.concatenate([jnp.sum(xn, 0)[None, :],
                                        jnp.sum(xn * xn, 0)[None, :]], axis=0)


def _node_tc(xb, c0m, c1m, c0e, c1e, batch2d, W2x_T, W2a_T, W2u_T, u, bn2,
             act):
    Fx = xb.shape[1]
    Fu = u.shape[1]
    full = lambda s: pl.BlockSpec(s, lambda i: (0, 0))
    return pl.pallas_call(
        functools.partial(_node_body, act),
        grid=(N_NODES // BN,),
        in_specs=[
            pl.BlockSpec((BN, Fx), lambda i: (i, 0)),
            pl.BlockSpec((BN, HID), lambda i: (i, 0)),
            pl.BlockSpec((BN, HID), lambda i: (i, 0)),
            pl.BlockSpec((BN, HID), lambda i: (i, 0)),
            pl.BlockSpec((BN, HID), lambda i: (i, 0)),
            pl.BlockSpec((BN, 1), lambda i: (i, 0)),
            full((Fx, HID)), full((HID, HID)), full((Fu, HID)),
            full((N_GRAPHS, Fu)), full((1, HID)),
        ],
        out_specs=[
            pl.BlockSpec((BN, HID), lambda i: (i, 0)),
            pl.BlockSpec((N_GRAPHS, HID), lambda i: (0, 0)),
            pl.BlockSpec((N_GRAPHS, HID), lambda i: (0, 0)),
            pl.BlockSpec((2, HID), lambda i: (0, 0)),
        ],
        out_shape=[
            jax.ShapeDtypeStruct((N_NODES, HID), _f32),
            jax.ShapeDtypeStruct((N_GRAPHS, HID), _f32),
            jax.ShapeDtypeStruct((N_GRAPHS, HID), _f32),
            jax.ShapeDtypeStruct((2, HID), _f32),
        ],
    )(xb, c0m, c1m, c0e, c1e, batch2d, W2x_T, W2a_T, W2u_T, u, bn2)


def _global_body(act, na_ref, ea_ref, u_ref, wg_ref, bg_ref, gam_ref,
                 bet_ref, un_ref):
    gin = jnp.concatenate([na_ref[...], ea_ref[...], u_ref[...]], axis=1)
    up = jnp.dot(gin, wg_ref[...], preferred_element_type=_f32) + bg_ref[...]
    if act:
        up = jnp.maximum(up, 0.0)
        m = jnp.mean(up, axis=0, keepdims=True)
        d = up - m
        v = jnp.mean(d * d, axis=0, keepdims=True)
        up = d * lax.rsqrt(v + 1e-5) * gam_ref[...] + bet_ref[...]
    un_ref[...] = up


def _global_tc(nagg, eagg, u, Wg_T, bg, gam, bet, act):
    Fu = u.shape[1]
    gdim = Wg_T.shape[0]
    full = lambda s: pl.BlockSpec(s, lambda i: (0, 0))
    return pl.pallas_call(
        functools.partial(_global_body, act),
        grid=(1,),
        in_specs=[full((N_GRAPHS, HID)), full((N_GRAPHS, HID)),
                  full((N_GRAPHS, Fu)), full((gdim, HID)), full((1, HID)),
                  full((1, HID)), full((1, HID))],
        out_specs=full((N_GRAPHS, HID)),
        out_shape=jax.ShapeDtypeStruct((N_GRAPHS, HID), _f32),
    )(nagg, eagg, u, Wg_T, bg, gam, bet)


def _sc_gather_body(ta_hbm, tb_hbm, col_hbm, row_hbm, g_hbm, ic, ir, ba, bb,
                    sa, sb):
    wid = lax.axis_index("s") * 2 + lax.axis_index("c")
    base = wid * EPW
    pltpu.sync_copy(col_hbm.at[wid], ic)
    pltpu.sync_copy(row_hbm.at[wid], ir)

    def chunk(j, carry):
        cpa = pltpu.async_copy(ta_hbm.at[ic.at[j]], ba, sa)
        cpb = pltpu.async_copy(tb_hbm.at[ir.at[j]], bb, sb)
        cpa.wait()
        cpb.wait()

        def addrow(r, c2):
            for k in range(8):
                sl = pl.ds(k * 16, 16)
                ba[r, sl] = ba[r, sl] + bb[r, sl]
            return c2
        lax.fori_loop(0, CB, addrow, 0, unroll=4)
        pltpu.sync_copy(ba, g_hbm.at[pl.ds(base + j * CB, CB)])
        return carry
    lax.fori_loop(0, NCH, chunk, 0)


def _sc_scatter_body(mm_hbm, me_hbm, col_hbm, seq_hbm, zr_hbm, cp_hbm,
                     ic80, iz80, mbm, mbe, sem, cshm, cshe):
    cc = lax.axis_index("c")
    ss = lax.axis_index("s")
    wid = ss * 2 + cc
    base = wid * EPW
    nzc = N_NODES // CB

    pltpu.sync_copy(zr_hbm, mbm)

    def zchunk(k, c2):
        c = jnp.minimum(ss * 8 + k, nzc - 1)
        pltpu.sync_copy(seq_hbm.at[c], iz80)
        pltpu.sync_copy(mbm, cshm.at[iz80])
        pltpu.sync_copy(mbm, cshe.at[iz80])
        return c2
    lax.fori_loop(0, 8, zchunk, 0)
    plsc.subcore_barrier()

    def chunk(j, carry):
        pltpu.sync_copy(col_hbm.at[wid, j], ic80)
        pltpu.sync_copy(mm_hbm.at[pl.ds(base + j * CB, CB)], mbm)
        pltpu.sync_copy(me_hbm.at[pl.ds(base + j * CB, CB)], mbe)
        pltpu.sync_copy(mbm, cshm.at[ic80], add=True)
        pltpu.sync_copy(mbe, cshe.at[ic80], add=True)
        return carry
    lax.fori_loop(0, NCH, chunk, 0)
    plsc.subcore_barrier()

    def dchunk(k, c2):
        c = jnp.minimum(ss * 8 + k, nzc - 1)
        sl = pl.ds(c * CB, CB)
        pltpu.sync_copy(seq_hbm.at[c], iz80)
        pltpu.async_copy(cshm.at[iz80], mbm, sem).wait()
        pltpu.sync_copy(mbm, cp_hbm.at[cc, 0, sl])
        pltpu.async_copy(cshe.at[iz80], mbe, sem).wait()
        pltpu.sync_copy(mbe, cp_hbm.at[cc, 1, sl])
        return c2
    lax.fori_loop(0, 8, dchunk, 0)


@functools.cache
def _sc_kernels():
    mesh = plsc.VectorSubcoreMesh(core_axis_name="c", subcore_axis_name="s")
    gather = functools.partial(
        pl.kernel, mesh=mesh,
        out_type=jax.ShapeDtypeStruct((N_EDGES, 128), _f32),
        scratch_types=[
            pltpu.VMEM((NCH, CB), jnp.int32),
            pltpu.VMEM((NCH, CB), jnp.int32),
            pltpu.VMEM((CB, 128), _f32),
            pltpu.VMEM((CB, 128), _f32),
            pltpu.SemaphoreType.DMA,
            pltpu.SemaphoreType.DMA,
        ])(_sc_gather_body)
    scatter = functools.partial(
        pl.kernel, mesh=mesh,
        out_type=jax.ShapeDtypeStruct((2, 2, N_NODES, HID), _f32),
        scratch_types=[
            pltpu.VMEM((CB,), jnp.int32),
            pltpu.VMEM((CB,), jnp.int32),
            pltpu.VMEM((CB, HID), _f32),
            pltpu.VMEM((CB, HID), _f32),
            pltpu.SemaphoreType.DMA,
            pltpu.VMEM_SHARED((N_NODES, HID), _f32),
            pltpu.VMEM_SHARED((N_NODES, HID), _f32),
        ])(_sc_scatter_body)
    return gather, scatter


def _gather_add(TA, TB, col3, row3):
    return _sc_kernels()[0](TA, TB, col3, row3)


def _scatter_partials(Mm, Me, col3):
    col = col3.reshape(-1)
    Cm = jax.ops.segment_sum(Mm, col, num_segments=N_NODES)
    Ce = jax.ops.segment_sum(Me, col, num_segments=N_NODES)
    z = jnp.zeros_like(Cm)
    return jnp.stack([jnp.stack([Cm, Ce]), jnp.stack([z, z])])


def kernel(x, edge_index, edge_attr, u, batch, params):
    row3 = edge_index[0].astype(jnp.int32).reshape(NW, NCH, CB)
    col3 = edge_index[1].astype(jnp.int32).reshape(NW, NCH, CB)
    batch2d = batch.astype(jnp.int32).reshape(N_NODES, 1)

    e_scale = None
    e_shift = None
    ss_x = jnp.concatenate([jnp.ones((1, x.shape[1]), _f32),
                            jnp.zeros((1, x.shape[1]), _f32)], axis=0)
    x_raw = x
    u_cur = u
    Eprev = edge_attr
    L = len(params)
    x_out = e_out = None

    for i, p in enumerate(params):
        act = i != L - 1
        Fx = x_raw.shape[1]
        Fe = Eprev.shape[1] if i == 0 else HID
        We, be = p['We'], p['be']
        Wn1, bn1 = p['Wn1'], p['bn1']
        Wn2, bn2 = p['Wn2'], p['bn2']
        Wg, bg = p['Wg'], p['bg']
        We_d, We_s = We[:, :Fx], We[:, Fx:2 * Fx]
        We_e, We_u = We[:, 2 * Fx:2 * Fx + Fe], We[:, 2 * Fx + Fe:]
        Wn1_d, Wn1_s = Wn1[:, :Fx], Wn1[:, Fx:2 * Fx]
        Wn1_e, Wn1_u = Wn1[:, 2 * Fx:2 * Fx + HID], Wn1[:, 2 * Fx + HID:]
        if e_scale is None:
            ss_e = jnp.concatenate([jnp.ones((1, Fe), _f32),
                                    jnp.zeros((1, Fe), _f32)], axis=0)
        else:
            ss_e = jnp.concatenate([e_scale[None, :], e_shift[None, :]],
                                   axis=0)
        WA_T = jnp.concatenate([We_d, Wn1_d], axis=0).T
        WB_T = jnp.concatenate([We_s, Wn1_s], axis=0).T
        cA = jnp.concatenate([be, bn1])[None, :]

        TA, TB, xb = _tables_tc(x_raw, batch2d, ss_x, WA_T, WB_T,
                                Wn1_u.T, We_u.T, cA, u_cur)
        G = _gather_add(TA, TB, col3, row3)
        Mm, Me, estats = _edge_tc(G, Eprev, ss_e, We_e.T, Wn1_e.T, act)
        Cp = _scatter_partials(Mm, Me, col3)
        x_raw, nagg, eagg, xstats = _node_tc(
            xb, Cp[0, 0], Cp[1, 0], Cp[0, 1], Cp[1, 1], batch2d,
            Wn2[:, :Fx].T, Wn2[:, Fx:Fx + HID].T, Wn2[:, Fx + HID:].T,
            u_cur, bn2[None, :], act)
        u_cur = _global_tc(nagg, eagg, u_cur, Wg.T, bg[None, :],
                           (p['gam_g'] if act else bg)[None, :],
                           (p['bet_g'] if act else bg)[None, :], act)
        if act:
            mx = xstats[0] / N_NODES
            vx = xstats[1] / N_NODES - mx * mx
            sx = p['gam_n'] / jnp.sqrt(vx + 1e-5)
            ss_x = jnp.concatenate([sx[None, :], (p['bet_n'] - mx * sx)[None, :]],
                                   axis=0)
            me = estats[0] / N_EDGES
            ve = estats[1] / N_EDGES - me * me
            e_scale = p['gam_e'] / jnp.sqrt(ve + 1e-5)
            e_shift = p['bet_e'] - me * e_scale
            Eprev = Me
        else:
            x_out = x_raw
            e_out = Me

    return (x_out, e_out, u_cur)
